# baseline (speedup 1.0000x reference)
import jax
import jax.numpy as jnp
from jax.experimental import pallas as pl
from jax.experimental.pallas import tpu as pltpu


_NCHUNKS = 32


def _copy_body(src_hbm, dst_hbm, sems):
    rows = src_hbm.shape[0]
    chunk = rows // _NCHUNKS
    for i in range(_NCHUNKS):
        sl = pl.ds(i * chunk, chunk)
        pltpu.make_async_copy(src_hbm.at[sl], dst_hbm.at[0, sl], sems.at[i]).start()
    for i in range(_NCHUNKS):
        sl = pl.ds(i * chunk, chunk)
        pltpu.make_async_copy(src_hbm.at[sl], dst_hbm.at[0, sl], sems.at[i]).wait()


def kernel(x, pe_weight):
    seq_len = x.shape[1]
    hidden = pe_weight.shape[1]
    return pl.pallas_call(
        _copy_body,
        out_shape=jax.ShapeDtypeStruct((1, seq_len, hidden), pe_weight.dtype),
        in_specs=[pl.BlockSpec(memory_space=pl.ANY)],
        out_specs=pl.BlockSpec(memory_space=pl.ANY),
        scratch_shapes=[pltpu.SemaphoreType.DMA((_NCHUNKS,))],
    )(pe_weight)


# speedup vs baseline: 41.2580x; 41.2580x over previous
import jax
import jax.numpy as jnp
from jax.experimental import pallas as pl
from jax.experimental.pallas import tpu as pltpu


_BLOCK_ROWS = 512


def _copy_body(src_ref, dst_ref):
    dst_ref[0] = src_ref[...]


def kernel(x, pe_weight):
    seq_len = x.shape[1]
    hidden = pe_weight.shape[1]
    grid = (seq_len // _BLOCK_ROWS,)
    return pl.pallas_call(
        _copy_body,
        out_shape=jax.ShapeDtypeStruct((1, seq_len, hidden), pe_weight.dtype),
        grid=grid,
        in_specs=[pl.BlockSpec((_BLOCK_ROWS, hidden), lambda i: (i, 0))],
        out_specs=pl.BlockSpec((1, _BLOCK_ROWS, hidden), lambda i: (0, i, 0)),
        compiler_params=pltpu.CompilerParams(
            dimension_semantics=("parallel",),
        ),
    )(pe_weight)


# speedup vs baseline: 45.0621x; 1.0922x over previous
import jax
import jax.numpy as jnp
from jax.experimental import pallas as pl
from jax.experimental.pallas import tpu as pltpu


_BLOCK_ROWS = 1024


def _copy_body(src_ref, dst_ref):
    dst_ref[0] = src_ref[...]


def kernel(x, pe_weight):
    seq_len = x.shape[1]
    hidden = pe_weight.shape[1]
    grid = (seq_len // _BLOCK_ROWS,)
    return pl.pallas_call(
        _copy_body,
        out_shape=jax.ShapeDtypeStruct((1, seq_len, hidden), pe_weight.dtype),
        grid=grid,
        in_specs=[pl.BlockSpec((_BLOCK_ROWS, hidden), lambda i: (i, 0))],
        out_specs=pl.BlockSpec((1, _BLOCK_ROWS, hidden), lambda i: (0, i, 0)),
        compiler_params=pltpu.CompilerParams(
            dimension_semantics=("parallel",),
        ),
    )(pe_weight)


# speedup vs baseline: 48.7795x; 1.0825x over previous
import jax
import jax.numpy as jnp
from jax.experimental import pallas as pl
from jax.experimental.pallas import tpu as pltpu


_BLOCK_ROWS = 2048


def _copy_body(src_ref, dst_ref):
    dst_ref[0] = src_ref[...]


def kernel(x, pe_weight):
    seq_len = x.shape[1]
    hidden = pe_weight.shape[1]
    grid = (seq_len // _BLOCK_ROWS,)
    return pl.pallas_call(
        _copy_body,
        out_shape=jax.ShapeDtypeStruct((1, seq_len, hidden), pe_weight.dtype),
        grid=grid,
        in_specs=[pl.BlockSpec((_BLOCK_ROWS, hidden), lambda i: (i, 0))],
        out_specs=pl.BlockSpec((1, _BLOCK_ROWS, hidden), lambda i: (0, i, 0)),
        compiler_params=pltpu.CompilerParams(
            dimension_semantics=("parallel",),
        ),
    )(pe_weight)
